# Initial kernel scaffold; baseline (speedup 1.0000x reference)
import jax
import jax.numpy as jnp
from jax.experimental import pallas as pl


def kernel(node_attrs, batch, ptr, M1_ref, M2_ref, M3_ref, Veff_ref, M1_pred, M2_pred, M3_pred, Veff_pred, M1_weight, M2_weight, M3_weight, Veff_weight):
    raise NotImplementedError("write your pallas kernel here")


# speedup vs baseline: 4.2972x; 4.2972x over previous
import functools

import jax
import jax.numpy as jnp
from jax import lax
from jax.experimental import pallas as pl
from jax.experimental.pallas import tpu as pltpu
from jax.experimental.pallas import tpu_sc as plsc

N = 100000
B = 2000
NW = 32
L = 16
CHUNK = (N // (NW * L)) * L
TAIL_VECS = (N - NW * CHUNK) // L
CHUNK_PAD = CHUNK + L
NVEC = CHUNK // L + 1
PB = B // L


def _sc_body(attrs_hbm, batch_hbm, ptr_hbm,
             m1r_hbm, m2r_hbm, m3r_hbm, vr_hbm,
             m1p_hbm, m2p_hbm, m3p_hbm, vp_hbm,
             out_hbm,
             attrs_v, batch_v, ptr_v, inv2_v, part_v,
             m1r_v, m2r_v, m3r_v, vr_v, m1p_v, m2p_v, m3p_v, vp_v,
             sem):
    wid = lax.axis_index("s") * 2 + lax.axis_index("c")
    base = wid * CHUNK
    iota = lax.iota(jnp.int32, L)
    zero_i = jnp.zeros((L,), jnp.int32)
    zero_f = jnp.zeros((L,), jnp.float32)

    pairs = [
        (attrs_hbm.at[pl.ds(base, CHUNK), :], attrs_v.at[pl.ds(0, CHUNK), :]),
        (batch_hbm.at[pl.ds(base, CHUNK)], batch_v.at[pl.ds(0, CHUNK)]),
        (m1r_hbm.at[pl.ds(base, CHUNK)], m1r_v.at[pl.ds(0, CHUNK)]),
        (m2r_hbm.at[pl.ds(base, CHUNK)], m2r_v.at[pl.ds(0, CHUNK)]),
        (m3r_hbm.at[pl.ds(base, CHUNK)], m3r_v.at[pl.ds(0, CHUNK)]),
        (vr_hbm.at[pl.ds(base, CHUNK)], vr_v.at[pl.ds(0, CHUNK)]),
        (m1p_hbm.at[pl.ds(base, CHUNK)], m1p_v.at[pl.ds(0, CHUNK)]),
        (m2p_hbm.at[pl.ds(base, CHUNK)], m2p_v.at[pl.ds(0, CHUNK)]),
        (m3p_hbm.at[pl.ds(base, CHUNK)], m3p_v.at[pl.ds(0, CHUNK)]),
        (vp_hbm.at[pl.ds(base, CHUNK)], vp_v.at[pl.ds(0, CHUNK)]),
    ]
    ptr_cp = pltpu.async_copy(ptr_hbm, ptr_v, sem)
    handles = [pltpu.async_copy(s, d, sem) for s, d in pairs]

    ptr_cp.wait()

    def pbody(k, _):
        off = k * L
        lo = ptr_v[pl.ds(off, L)]
        hi = plsc.load_gather(ptr_v, [off + 1 + iota])
        c = (hi - lo).astype(jnp.float32)
        inv2_v[pl.ds(off, L)] = 1.0 / (c * c)
        return 0

    lax.fori_loop(0, PB, pbody, 0)

    for h in handles:
        h.wait()

    tail = N - TAIL_VECS * L + wid * L

    @pl.when(wid < TAIL_VECS)
    def _():
        pltpu.sync_copy(attrs_hbm.at[pl.ds(tail, L), :],
                        attrs_v.at[pl.ds(CHUNK, L), :])
        pltpu.sync_copy(batch_hbm.at[pl.ds(tail, L)],
                        batch_v.at[pl.ds(CHUNK, L)])
        pltpu.sync_copy(m1r_hbm.at[pl.ds(tail, L)], m1r_v.at[pl.ds(CHUNK, L)])
        pltpu.sync_copy(m2r_hbm.at[pl.ds(tail, L)], m2r_v.at[pl.ds(CHUNK, L)])
        pltpu.sync_copy(m3r_hbm.at[pl.ds(tail, L)], m3r_v.at[pl.ds(CHUNK, L)])
        pltpu.sync_copy(vr_hbm.at[pl.ds(tail, L)], vr_v.at[pl.ds(CHUNK, L)])
        pltpu.sync_copy(m1p_hbm.at[pl.ds(tail, L)], m1p_v.at[pl.ds(CHUNK, L)])
        pltpu.sync_copy(m2p_hbm.at[pl.ds(tail, L)], m2p_v.at[pl.ds(CHUNK, L)])
        pltpu.sync_copy(m3p_hbm.at[pl.ds(tail, L)], m3p_v.at[pl.ds(CHUNK, L)])
        pltpu.sync_copy(vp_hbm.at[pl.ds(tail, L)], vp_v.at[pl.ds(CHUNK, L)])

    @pl.when(wid >= TAIL_VECS)
    def _():
        plsc.store_scatter(attrs_v, [CHUNK + iota, zero_i], zero_f)
        batch_v[pl.ds(CHUNK, L)] = zero_i
        m1r_v[pl.ds(CHUNK, L)] = zero_f
        m2r_v[pl.ds(CHUNK, L)] = zero_f
        m3r_v[pl.ds(CHUNK, L)] = zero_f
        vr_v[pl.ds(CHUNK, L)] = zero_f
        m1p_v[pl.ds(CHUNK, L)] = zero_f
        m2p_v[pl.ds(CHUNK, L)] = zero_f
        m3p_v[pl.ds(CHUNK, L)] = zero_f
        vp_v[pl.ds(CHUNK, L)] = zero_f

    def body(k, carry):
        a1, a2, a3, a4, cn = carry
        off = k * L
        b = batch_v[pl.ds(off, L)]
        g = plsc.load_gather(inv2_v, [b])
        a0 = plsc.load_gather(attrs_v, [off + iota, zero_i])
        m = a0 > 0.5
        gm = jnp.where(m, g, 0.0)
        cn = cn + jnp.where(m, 1.0, 0.0)
        d1 = m1r_v[pl.ds(off, L)] - m1p_v[pl.ds(off, L)]
        d2 = m2r_v[pl.ds(off, L)] - m2p_v[pl.ds(off, L)]
        d3 = m3r_v[pl.ds(off, L)] - m3p_v[pl.ds(off, L)]
        d4 = vr_v[pl.ds(off, L)] - vp_v[pl.ds(off, L)]
        a1 = a1 + gm * d1 * d1
        a2 = a2 + gm * d2 * d2
        a3 = a3 + gm * d3 * d3
        a4 = a4 + gm * d4 * d4
        return a1, a2, a3, a4, cn

    a1, a2, a3, a4, cn = lax.fori_loop(
        0, NVEC, body, (zero_f, zero_f, zero_f, zero_f, zero_f))

    part_v[pl.ds(0, L)] = a1
    part_v[pl.ds(L, L)] = a2
    part_v[pl.ds(2 * L, L)] = a3
    part_v[pl.ds(3 * L, L)] = a4
    part_v[pl.ds(4 * L, L)] = cn
    pltpu.sync_copy(part_v, out_hbm.at[wid])


_sc_partials = functools.partial(
    pl.kernel,
    out_type=jax.ShapeDtypeStruct((NW, 5 * L), jnp.float32),
    mesh=plsc.VectorSubcoreMesh(core_axis_name="c", subcore_axis_name="s"),
    compiler_params=pltpu.CompilerParams(
        needs_layout_passes=False, use_tc_tiling_on_sc=False),
    scratch_types=[
        pltpu.VMEM((CHUNK_PAD, 7), jnp.float32),
        pltpu.VMEM((CHUNK_PAD,), jnp.int32),
        pltpu.VMEM((B + 1,), jnp.int32),
        pltpu.VMEM((B,), jnp.float32),
        pltpu.VMEM((5 * L,), jnp.float32),
        pltpu.VMEM((CHUNK_PAD,), jnp.float32),
        pltpu.VMEM((CHUNK_PAD,), jnp.float32),
        pltpu.VMEM((CHUNK_PAD,), jnp.float32),
        pltpu.VMEM((CHUNK_PAD,), jnp.float32),
        pltpu.VMEM((CHUNK_PAD,), jnp.float32),
        pltpu.VMEM((CHUNK_PAD,), jnp.float32),
        pltpu.VMEM((CHUNK_PAD,), jnp.float32),
        pltpu.VMEM((CHUNK_PAD,), jnp.float32),
        pltpu.SemaphoreType.DMA,
    ],
)(_sc_body)


def _fin_body(parts_ref, w_ref, o_ref):
    x = parts_ref[...]
    s1 = jnp.sum(x[:, 0:L])
    s2 = jnp.sum(x[:, L:2 * L])
    s3 = jnp.sum(x[:, 2 * L:3 * L])
    s4 = jnp.sum(x[:, 3 * L:4 * L])
    cnt = jnp.sum(x[:, 4 * L:5 * L])
    tot = (w_ref[0] * s1 + w_ref[1] * s2 + w_ref[2] * s3 + w_ref[3] * s4) / cnt
    o_ref[0, 0] = jnp.where(cnt > 0, tot, 0.0)


_finalize = pl.pallas_call(
    _fin_body,
    out_shape=jax.ShapeDtypeStruct((1, 1), jnp.float32),
    in_specs=[
        pl.BlockSpec(memory_space=pltpu.VMEM),
        pl.BlockSpec(memory_space=pltpu.SMEM),
    ],
    out_specs=pl.BlockSpec(memory_space=pltpu.SMEM),
)


def kernel(node_attrs, batch, ptr, M1_ref, M2_ref, M3_ref, Veff_ref,
           M1_pred, M2_pred, M3_pred, Veff_pred,
           M1_weight, M2_weight, M3_weight, Veff_weight):
    batch = batch.astype(jnp.int32)
    ptr = ptr.astype(jnp.int32)
    parts = _sc_partials(node_attrs, batch, ptr,
                         M1_ref, M2_ref, M3_ref, Veff_ref,
                         M1_pred, M2_pred, M3_pred, Veff_pred)
    w = jnp.stack([M1_weight, M2_weight, M3_weight, Veff_weight]).astype(
        jnp.float32)
    out = _finalize(parts, w)
    return out[0, 0]


# speedup vs baseline: 6.4277x; 1.4958x over previous
import functools

import jax
import jax.numpy as jnp
from jax import lax
from jax.experimental import pallas as pl
from jax.experimental.pallas import tpu as pltpu
from jax.experimental.pallas import tpu_sc as plsc

N = 100000
B = 2000
NW = 32
L = 16
CHUNK = (N // (NW * L)) * L
TAIL_VECS = (N - NW * CHUNK) // L
CHUNK_PAD = CHUNK + L
NVEC = CHUNK // L + 1
PB = B // L

BLK = 5120
ASTEPS = -(-N // BLK)


def _dense_body(w_ref, a_ref, m1r, m2r, m3r, vr, m1p, m2p, m3p, vp,
                p_ref, cnt_ref):
    i = pl.program_id(0)
    a0 = a_ref[:, 0:1]
    m = (a0 > 0.5).astype(jnp.float32)
    mf = m.reshape((BLK,))
    pos = lax.broadcasted_iota(jnp.int32, (BLK,), 0) + i * BLK
    mf = jnp.where(pos < N, mf, 0.0)
    d1 = m1r[...] - m1p[...]
    d2 = m2r[...] - m2p[...]
    d3 = m3r[...] - m3p[...]
    d4 = vr[...] - vp[...]
    q = (w_ref[0] * (d1 * d1) + w_ref[1] * (d2 * d2)
         + w_ref[2] * (d3 * d3) + w_ref[3] * (d4 * d4))
    p_ref[...] = q * mf

    @pl.when(i == 0)
    def _():
        cnt_ref[0, 0] = 0.0

    cnt_ref[0, 0] += jnp.sum(mf)


_dense = pl.pallas_call(
    _dense_body,
    grid=(ASTEPS,),
    in_specs=[
        pl.BlockSpec(memory_space=pltpu.SMEM),
        pl.BlockSpec((BLK, 7), lambda i: (i, 0)),
    ] + [pl.BlockSpec((BLK,), lambda i: (i,))] * 8,
    out_specs=[
        pl.BlockSpec((BLK,), lambda i: (i,)),
        pl.BlockSpec(memory_space=pltpu.SMEM),
    ],
    out_shape=[
        jax.ShapeDtypeStruct((N,), jnp.float32),
        jax.ShapeDtypeStruct((1, 1), jnp.float32),
    ],
)


def _sc_body(batch_hbm, ptr_hbm, p_hbm, out_hbm,
             batch_v, ptr_v, inv2_v, p_v, part_v, sem):
    wid = lax.axis_index("s") * 2 + lax.axis_index("c")
    base = wid * CHUNK
    iota = lax.iota(jnp.int32, L)
    zero_i = jnp.zeros((L,), jnp.int32)
    zero_f = jnp.zeros((L,), jnp.float32)

    ptr_cp = pltpu.async_copy(ptr_hbm, ptr_v, sem)
    b_cp = pltpu.async_copy(batch_hbm.at[pl.ds(base, CHUNK)],
                            batch_v.at[pl.ds(0, CHUNK)], sem)
    p_cp = pltpu.async_copy(p_hbm.at[pl.ds(base, CHUNK)],
                            p_v.at[pl.ds(0, CHUNK)], sem)

    ptr_cp.wait()

    def pbody(k, _):
        off = k * L
        lo = ptr_v[pl.ds(off, L)]
        hi = plsc.load_gather(ptr_v, [off + 1 + iota])
        c = (hi - lo).astype(jnp.float32)
        inv2_v[pl.ds(off, L)] = 1.0 / (c * c)
        return 0

    lax.fori_loop(0, PB, pbody, 0)

    b_cp.wait()
    p_cp.wait()

    tail = N - TAIL_VECS * L + wid * L

    @pl.when(wid < TAIL_VECS)
    def _():
        pltpu.sync_copy(batch_hbm.at[pl.ds(tail, L)],
                        batch_v.at[pl.ds(CHUNK, L)])
        pltpu.sync_copy(p_hbm.at[pl.ds(tail, L)], p_v.at[pl.ds(CHUNK, L)])

    @pl.when(wid >= TAIL_VECS)
    def _():
        batch_v[pl.ds(CHUNK, L)] = zero_i
        p_v[pl.ds(CHUNK, L)] = zero_f

    def body(k, acc):
        off = k * L
        b = batch_v[pl.ds(off, L)]
        g = plsc.load_gather(inv2_v, [b])
        return acc + p_v[pl.ds(off, L)] * g

    acc = lax.fori_loop(0, NVEC, body, zero_f)

    part_v[pl.ds(0, L)] = acc
    pltpu.sync_copy(part_v, out_hbm.at[wid])


_sc_gather_reduce = functools.partial(
    pl.kernel,
    out_type=jax.ShapeDtypeStruct((NW, L), jnp.float32),
    mesh=plsc.VectorSubcoreMesh(core_axis_name="c", subcore_axis_name="s"),
    compiler_params=pltpu.CompilerParams(
        needs_layout_passes=False, use_tc_tiling_on_sc=False),
    scratch_types=[
        pltpu.VMEM((CHUNK_PAD,), jnp.int32),
        pltpu.VMEM((B + 1,), jnp.int32),
        pltpu.VMEM((B,), jnp.float32),
        pltpu.VMEM((CHUNK_PAD,), jnp.float32),
        pltpu.VMEM((L,), jnp.float32),
        pltpu.SemaphoreType.DMA,
    ],
)(_sc_body)


def _fin_body(parts_ref, cnt_ref, o_ref):
    s = jnp.sum(parts_ref[...])
    cnt = cnt_ref[0, 0]
    o_ref[0, 0] = jnp.where(cnt > 0, s / cnt, 0.0)


_finalize = pl.pallas_call(
    _fin_body,
    out_shape=jax.ShapeDtypeStruct((1, 1), jnp.float32),
    in_specs=[
        pl.BlockSpec(memory_space=pltpu.VMEM),
        pl.BlockSpec(memory_space=pltpu.SMEM),
    ],
    out_specs=pl.BlockSpec(memory_space=pltpu.SMEM),
)


def kernel(node_attrs, batch, ptr, M1_ref, M2_ref, M3_ref, Veff_ref,
           M1_pred, M2_pred, M3_pred, Veff_pred,
           M1_weight, M2_weight, M3_weight, Veff_weight):
    batch = batch.astype(jnp.int32)
    ptr = ptr.astype(jnp.int32)
    w = jnp.stack([M1_weight, M2_weight, M3_weight, Veff_weight]).astype(
        jnp.float32)
    p, cnt = _dense(w, node_attrs, M1_ref, M2_ref, M3_ref, Veff_ref,
                    M1_pred, M2_pred, M3_pred, Veff_pred)
    parts = _sc_gather_reduce(batch, ptr, p)
    out = _finalize(parts, cnt)
    return out[0, 0]
